# baseline (speedup 1.0000x reference)
import jax
import jax.numpy as jnp
from jax import lax
from jax.experimental import pallas as pl
from jax.experimental.pallas import tpu as pltpu

U = 196
UP = 208
TOPK = 64
NH = 4
HD = 64
TOK_BLK = 512


def _b16(x):
    return x.astype(jnp.bfloat16)


def _bdot(x, y):
    return jnp.dot(_b16(x), _b16(y), preferred_element_type=jnp.float32)


def _bdot_t(x, y):
    return lax.dot_general(_b16(x), _b16(y), (((1,), (1,)), ((), ())),
                           preferred_element_type=jnp.float32)


def _k1_body(x_ref, w1t_ref, b1_ref, w2t_ref, b2_ref, lnkg_ref, lnkb_ref,
             wkt_ref, bk_ref, wvt_ref, bv_ref, a_ref, kp_ref, vp_ref):
    x = x_ref[...]
    h = _bdot(x, w1t_ref[...]) + b1_ref[...]
    h = jnp.where(h > 0, h, 0.01 * h)
    a = _bdot(h, w2t_ref[...]) + b2_ref[...]
    a_ref[...] = a
    mu = jnp.mean(a, axis=-1, keepdims=True)
    var = jnp.mean((a - mu) * (a - mu), axis=-1, keepdims=True)
    ln = (a - mu) / jnp.sqrt(var + 1e-5) * lnkg_ref[...] + lnkb_ref[...]
    kp_ref[...] = _bdot(ln, wkt_ref[...]) + bk_ref[...]
    vp_ref[...] = _bdot(ln, wvt_ref[...]) + bv_ref[...]


def _k2_body(ids_ref, idx_ref, perm_ref, a_ref, kp_ref, vp_ref,
             lnqg_ref, lnqb_ref, wqt_ref, bq_ref, owt_ref, ob_ref, out_ref):
    m = pl.program_id(0)
    idx = idx_ref[m]

    a_blk = a_ref[0]
    q = a_ref[0, pl.ds(idx, 1), :]

    qn = q / jnp.maximum(jnp.sqrt(jnp.sum(q * q)), 1e-12)
    anorm = jnp.sqrt(jnp.sum(a_blk * a_blk, axis=1, keepdims=True))
    an = a_blk / jnp.maximum(anorm, 1e-12)
    sim = _bdot_t(qn, an)

    col = lax.broadcasted_iota(jnp.int32, (1, UP), 1)
    valid = col < U
    simv = jnp.where(valid, sim, -3.0)

    key = lax.bitcast_convert_type(simv, jnp.int32)
    key = jnp.where(key < 0, key ^ jnp.int32(0x7FFFFFFF), key)
    int_min = jnp.int32(-2147483648)

    def bit_step(i, p):
        test = p | (jnp.int32(1) << (jnp.int32(31) - i))
        cnt = jnp.sum((key >= (test ^ int_min)).astype(jnp.int32))
        return jnp.where(cnt >= TOPK, test, p)

    p = lax.fori_loop(0, 32, bit_step, jnp.int32(0))
    selected = (key >= (p ^ int_min)) & valid

    mu = jnp.mean(q)
    var = jnp.mean((q - mu) * (q - mu))
    lnq = (q - mu) / jnp.sqrt(var + 1e-5) * lnqg_ref[...] + lnqb_ref[...]
    qp = _bdot(lnq, wqt_ref[...]) + bq_ref[...]

    kp_blk = kp_ref[0]
    vp_blk = vp_ref[0]
    scale = 1.0 / jnp.sqrt(jnp.float32(HD))
    ctx_cols = []
    for h in range(NH):
        sl = slice(h * HD, (h + 1) * HD)
        s = _bdot_t(qp[:, sl], kp_blk[:, sl]) * scale
        s = jnp.where(selected, s, -1e30)
        mx = jnp.max(s, axis=1, keepdims=True)
        e = jnp.where(selected, jnp.exp(s - mx), 0.0)
        attn = e / jnp.sum(e, axis=1, keepdims=True)
        ctx_cols.append(_bdot(attn, vp_blk[:, sl]))
    ctx = jnp.concatenate(ctx_cols, axis=1)

    out_ref[0] = _bdot(ctx, owt_ref[...]) + ob_ref[...]


def kernel(patches, patch_ids, valid_mask, patch_center_gps, offsets,
           W1, b1, W2, b2, lnq_g, lnq_b, lnk_g, lnk_b, in_w, in_b, out_w, out_b):
    N, u, Din = patches.shape
    M = patch_ids.shape[0]
    D = W2.shape[0]
    hid = W1.shape[0]

    hg = int(u ** 0.5)
    dx = offsets[:, 0]
    dy = offsets[:, 1]
    i_t = jnp.clip(hg // 2 + dy, 0, hg - 1)
    j_t = jnp.clip(hg // 2 + dx, 0, hg - 1)
    idx_flat = (i_t * hg + j_t).astype(jnp.int32)
    ids = patch_ids.astype(jnp.int32)

    order = jnp.argsort(ids).astype(jnp.int32)
    ids_s = ids[order]
    idx_s = idx_flat[order]

    xpad = jnp.pad(patches, ((0, 0), (0, UP - u), (0, 0)))
    tokens = xpad.reshape(N * UP, Din)
    n_tok = N * UP

    w1t = W1.T
    w2t = W2.T
    wq, wk, wv = in_w[:D], in_w[D:2 * D], in_w[2 * D:]
    bq, bk, bv = in_b[:D], in_b[D:2 * D], in_b[2 * D:]
    row2 = lambda v: v.reshape(1, -1)

    n_blk = n_tok // TOK_BLK
    full2 = lambda r, c: pl.BlockSpec((r, c), lambda t: (0, 0))
    a_all, kp_all, vp_all = pl.pallas_call(
        _k1_body,
        grid=(n_blk,),
        in_specs=[
            pl.BlockSpec((TOK_BLK, Din), lambda t: (t, 0)),
            full2(Din, hid), full2(1, hid),
            full2(hid, D), full2(1, D),
            full2(1, D), full2(1, D),
            full2(D, D), full2(1, D),
            full2(D, D), full2(1, D),
        ],
        out_specs=[
            pl.BlockSpec((TOK_BLK, D), lambda t: (t, 0)),
            pl.BlockSpec((TOK_BLK, D), lambda t: (t, 0)),
            pl.BlockSpec((TOK_BLK, D), lambda t: (t, 0)),
        ],
        out_shape=[jax.ShapeDtypeStruct((n_tok, D), jnp.float32)] * 3,
    )(tokens, w1t, row2(b1), w2t, row2(b2), row2(lnk_g), row2(lnk_b),
      wk.T, row2(bk), wv.T, row2(bv))

    a3 = a_all.reshape(N, UP, D)
    kp3 = kp_all.reshape(N, UP, D)
    vp3 = vp_all.reshape(N, UP, D)

    gat = pl.BlockSpec((1, UP, D), lambda m, ids_r, idx_r, perm_r: (ids_r[m], 0, 0))
    cst = lambda r, c: pl.BlockSpec((r, c), lambda m, ids_r, idx_r, perm_r: (0, 0))
    grid_spec = pltpu.PrefetchScalarGridSpec(
        num_scalar_prefetch=3,
        grid=(M,),
        in_specs=[
            gat, gat, gat,
            cst(1, D), cst(1, D),
            cst(D, D), cst(1, D),
            cst(D, D), cst(1, D),
        ],
        out_specs=pl.BlockSpec(
            (1, 1, D), lambda m, ids_r, idx_r, perm_r: (perm_r[m], 0, 0)),
    )
    attn_out = pl.pallas_call(
        _k2_body,
        grid_spec=grid_spec,
        out_shape=jax.ShapeDtypeStruct((M, 1, D), jnp.float32),
    )(ids_s, idx_s, order, a3, kp3, vp3,
      row2(lnq_g), row2(lnq_b), wq.T, row2(bq), out_w.T, row2(out_b))
    attn_out = attn_out.reshape(M, D)

    B, T = valid_mask.shape
    flat_mask = valid_mask.reshape(-1)
    rank = jnp.cumsum(flat_mask.astype(jnp.int32)) - 1
    placed = attn_out[jnp.clip(rank, 0, M - 1)]
    return jnp.where(flat_mask[:, None], placed,
                     jnp.zeros((), dtype=attn_out.dtype)).reshape(B, T, D)


# speedup vs baseline: 4.5755x; 4.5755x over previous
import jax
import jax.numpy as jnp
from jax import lax
from jax.experimental import pallas as pl
from jax.experimental.pallas import tpu as pltpu

U = 196
UP = 208
TOPK = 64
NH = 4
HD = 64
TOK_BLK = 512
RB = 8


def _b16(x):
    return x.astype(jnp.bfloat16)


def _bdot(x, y):
    return jnp.dot(_b16(x), _b16(y), preferred_element_type=jnp.float32)


def _bdot_t(x, y):
    return lax.dot_general(_b16(x), _b16(y), (((1,), (1,)), ((), ())),
                           preferred_element_type=jnp.float32)


def _k1_body(x_ref, w1t_ref, b1_ref, w2t_ref, b2_ref, lnkg_ref, lnkb_ref,
             wkt_ref, bk_ref, wvt_ref, bv_ref, a_ref, kp_ref, vp_ref):
    x = x_ref[...]
    h = _bdot(x, w1t_ref[...]) + b1_ref[...]
    h = jnp.where(h > 0, h, 0.01 * h)
    a = _bdot(h, w2t_ref[...]) + b2_ref[...]
    a_ref[...] = a
    mu = jnp.mean(a, axis=-1, keepdims=True)
    var = jnp.mean((a - mu) * (a - mu), axis=-1, keepdims=True)
    ln = (a - mu) / jnp.sqrt(var + 1e-5) * lnkg_ref[...] + lnkb_ref[...]
    kp_ref[...] = _bdot(ln, wkt_ref[...]) + bk_ref[...]
    vp_ref[...] = _bdot(ln, wvt_ref[...]) + bv_ref[...]


def _k2_body(ids_ref, idx_ref, *refs):
    a_refs = refs[:RB]
    kp_refs = refs[RB:2 * RB]
    vp_refs = refs[2 * RB:3 * RB]
    (lnqg_ref, lnqb_ref, wqt_ref, bq_ref, owt_ref, ob_ref, out_ref) = refs[3 * RB:]
    m = pl.program_id(0)

    q_rows = []
    sim_rows = []
    for j in range(RB):
        idx = idx_ref[m * RB + j]
        a_blk = a_refs[j][0]
        q = a_refs[j][0, pl.ds(idx, 1), :]
        q_rows.append(q)
        qn = q / jnp.maximum(jnp.sqrt(jnp.sum(q * q)), 1e-12)
        anorm = jnp.sqrt(jnp.sum(a_blk * a_blk, axis=1, keepdims=True))
        an = a_blk / jnp.maximum(anorm, 1e-12)
        sim_rows.append(_bdot_t(qn, an))
    sim = jnp.concatenate(sim_rows, axis=0)
    qmat = jnp.concatenate(q_rows, axis=0)

    col = lax.broadcasted_iota(jnp.int32, (RB, UP), 1)
    valid = col < U
    simv = jnp.where(valid, sim, -3.0)

    key = lax.bitcast_convert_type(simv, jnp.int32)
    key = jnp.where(key < 0, key ^ jnp.int32(0x7FFFFFFF), key)
    int_min = jnp.int32(-2147483648)

    def bit_step(i, p):
        test = p | (jnp.int32(1) << (jnp.int32(31) - i))
        cnt = jnp.sum((key >= (test ^ int_min)).astype(jnp.int32),
                      axis=1, keepdims=True)
        return jnp.where(cnt >= TOPK, test, p)

    p = lax.fori_loop(0, 32, bit_step, jnp.zeros((RB, 1), jnp.int32))
    selected = (key >= (p ^ int_min)) & valid

    mu = jnp.mean(qmat, axis=1, keepdims=True)
    var = jnp.mean((qmat - mu) * (qmat - mu), axis=1, keepdims=True)
    lnq = (qmat - mu) / jnp.sqrt(var + 1e-5) * lnqg_ref[...] + lnqb_ref[...]
    qp = _bdot(lnq, wqt_ref[...]) + bq_ref[...]

    hrow = lax.broadcasted_iota(jnp.int32, (NH, 256), 0)
    hcol = lax.broadcasted_iota(jnp.int32, (NH, 256), 1)
    hmask = (hcol // HD == hrow).astype(jnp.float32)

    scale = 1.0 / jnp.sqrt(jnp.float32(HD))
    ctx_rows = []
    for j in range(RB):
        kp_blk = kp_refs[j][0]
        vp_blk = vp_refs[j][0]
        qmat_h = jnp.broadcast_to(qp[j:j + 1], (NH, 256)) * hmask
        s = _bdot_t(qmat_h, kp_blk) * scale
        sel = jnp.broadcast_to(selected[j:j + 1], (NH, UP))
        s = jnp.where(sel, s, -1e30)
        mx = jnp.max(s, axis=1, keepdims=True)
        e = jnp.where(sel, jnp.exp(s - mx), 0.0)
        attn = e / jnp.sum(e, axis=1, keepdims=True)
        c = _bdot(attn, vp_blk)
        ctx_rows.append(jnp.sum(c * hmask, axis=0, keepdims=True))
    ctx = jnp.concatenate(ctx_rows, axis=0)

    out_ref[...] = _bdot(ctx, owt_ref[...]) + ob_ref[...]


def kernel(patches, patch_ids, valid_mask, patch_center_gps, offsets,
           W1, b1, W2, b2, lnq_g, lnq_b, lnk_g, lnk_b, in_w, in_b, out_w, out_b):
    N, u, Din = patches.shape
    M = patch_ids.shape[0]
    D = W2.shape[0]
    hid = W1.shape[0]

    hg = int(u ** 0.5)
    dx = offsets[:, 0]
    dy = offsets[:, 1]
    i_t = jnp.clip(hg // 2 + dy, 0, hg - 1)
    j_t = jnp.clip(hg // 2 + dx, 0, hg - 1)
    idx_flat = (i_t * hg + j_t).astype(jnp.int32)
    ids = patch_ids.astype(jnp.int32)

    xpad = jnp.pad(patches, ((0, 0), (0, UP - u), (0, 0)))
    tokens = xpad.reshape(N * UP, Din)
    n_tok = N * UP

    w1t = W1.T
    w2t = W2.T
    wq, wk, wv = in_w[:D], in_w[D:2 * D], in_w[2 * D:]
    bq, bk, bv = in_b[:D], in_b[D:2 * D], in_b[2 * D:]
    row2 = lambda v: v.reshape(1, -1)

    n_blk = n_tok // TOK_BLK
    full2 = lambda r, c: pl.BlockSpec((r, c), lambda t: (0, 0))
    a_all, kp_all, vp_all = pl.pallas_call(
        _k1_body,
        grid=(n_blk,),
        in_specs=[
            pl.BlockSpec((TOK_BLK, Din), lambda t: (t, 0)),
            full2(Din, hid), full2(1, hid),
            full2(hid, D), full2(1, D),
            full2(1, D), full2(1, D),
            full2(D, D), full2(1, D),
            full2(D, D), full2(1, D),
        ],
        out_specs=[
            pl.BlockSpec((TOK_BLK, D), lambda t: (t, 0)),
            pl.BlockSpec((TOK_BLK, D), lambda t: (t, 0)),
            pl.BlockSpec((TOK_BLK, D), lambda t: (t, 0)),
        ],
        out_shape=[jax.ShapeDtypeStruct((n_tok, D), jnp.float32)] * 3,
    )(tokens, w1t, row2(b1), w2t, row2(b2), row2(lnk_g), row2(lnk_b),
      wk.T, row2(bk), wv.T, row2(bv))

    a3 = a_all.reshape(N, UP, D)
    kp3 = kp_all.reshape(N, UP, D)
    vp3 = vp_all.reshape(N, UP, D)

    def gat(j):
        return pl.BlockSpec(
            (1, UP, D), lambda m, ids_r, idx_r, j=j: (ids_r[m * RB + j], 0, 0))
    cst = lambda r, c: pl.BlockSpec((r, c), lambda m, ids_r, idx_r: (0, 0))
    grid_spec = pltpu.PrefetchScalarGridSpec(
        num_scalar_prefetch=2,
        grid=(M // RB,),
        in_specs=(
            [gat(j) for j in range(RB)] * 3 +
            [cst(1, D), cst(1, D), cst(D, D), cst(1, D), cst(D, D), cst(1, D)]
        ),
        out_specs=pl.BlockSpec((RB, D), lambda m, ids_r, idx_r: (m, 0)),
    )
    attn_out = pl.pallas_call(
        _k2_body,
        grid_spec=grid_spec,
        out_shape=jax.ShapeDtypeStruct((M, D), jnp.float32),
    )(ids, idx_flat,
      *([a3] * RB), *([kp3] * RB), *([vp3] * RB),
      row2(lnq_g), row2(lnq_b), wq.T, row2(bq), out_w.T, row2(out_b))

    B, T = valid_mask.shape
    flat_mask = valid_mask.reshape(-1)
    rank = jnp.cumsum(flat_mask.astype(jnp.int32)) - 1
    placed = attn_out[jnp.clip(rank, 0, M - 1)]
    return jnp.where(flat_mask[:, None], placed,
                     jnp.zeros((), dtype=attn_out.dtype)).reshape(B, T, D)


# speedup vs baseline: 4.5886x; 1.0029x over previous
import jax
import jax.numpy as jnp
from jax import lax
from jax.experimental import pallas as pl
from jax.experimental.pallas import tpu as pltpu

U = 196
TOPK = 64
NH = 4
HD = 64
PB = 4
RB = 8


def _b16(x):
    return x.astype(jnp.bfloat16)


def _bdot(x, y):
    return jnp.dot(_b16(x), _b16(y), preferred_element_type=jnp.float32)


def _bdot_t(x, y):
    return lax.dot_general(_b16(x), _b16(y), (((1,), (1,)), ((), ())),
                           preferred_element_type=jnp.float32)


def _k1_body(x_ref, w1_ref, b1_ref, w2_ref, b2_ref, lnkg_ref, lnkb_ref,
             wk_ref, bk_ref, wv_ref, bv_ref, a_ref, kp_ref, vp_ref):
    for i in range(PB):
        x = x_ref[i]
        h = _bdot_t(x, w1_ref[...]) + b1_ref[...]
        h = jnp.where(h > 0, h, 0.01 * h)
        a = _bdot_t(h, w2_ref[...]) + b2_ref[...]
        a_ref[i] = a
        mu = jnp.mean(a, axis=-1, keepdims=True)
        var = jnp.mean((a - mu) * (a - mu), axis=-1, keepdims=True)
        ln = (a - mu) / jnp.sqrt(var + 1e-5) * lnkg_ref[...] + lnkb_ref[...]
        kp_ref[i] = _b16(_bdot_t(ln, wk_ref[...]) + bk_ref[...])
        vp_ref[i] = _b16(_bdot_t(ln, wv_ref[...]) + bv_ref[...])


def _k2_body(ids_ref, idx_ref, *refs):
    a_refs = refs[:RB]
    kp_refs = refs[RB:2 * RB]
    vp_refs = refs[2 * RB:3 * RB]
    (lnqg_ref, lnqb_ref, wq_ref, bq_ref, ow_ref, ob_ref, out_ref) = refs[3 * RB:]
    m = pl.program_id(0)

    q_rows = []
    sim_rows = []
    for j in range(RB):
        idx = idx_ref[m * RB + j]
        a_blk = a_refs[j][0]
        q = a_refs[j][0, pl.ds(idx, 1), :]
        q_rows.append(q)
        qn = q / jnp.maximum(jnp.sqrt(jnp.sum(q * q)), 1e-12)
        anorm = jnp.sqrt(jnp.sum(a_blk * a_blk, axis=1, keepdims=True))
        an = a_blk / jnp.maximum(anorm, 1e-12)
        sim_rows.append(_bdot_t(qn, an))
    sim = jnp.concatenate(sim_rows, axis=0)
    qmat = jnp.concatenate(q_rows, axis=0)

    key = lax.bitcast_convert_type(sim, jnp.int32)
    key = jnp.where(key < 0, key ^ jnp.int32(0x7FFFFFFF), key)
    int_min = jnp.int32(-2147483648)

    def bit_step(i, p):
        test = p | (jnp.int32(1) << (jnp.int32(31) - i))
        cnt = jnp.sum((key >= (test ^ int_min)).astype(jnp.int32),
                      axis=1, keepdims=True)
        return jnp.where(cnt >= TOPK, test, p)

    p = lax.fori_loop(0, 32, bit_step, jnp.zeros((RB, 1), jnp.int32))
    selected = key >= (p ^ int_min)

    mu = jnp.mean(qmat, axis=1, keepdims=True)
    var = jnp.mean((qmat - mu) * (qmat - mu), axis=1, keepdims=True)
    lnq = (qmat - mu) / jnp.sqrt(var + 1e-5) * lnqg_ref[...] + lnqb_ref[...]
    qp = _bdot_t(lnq, wq_ref[...]) + bq_ref[...]

    hrow = lax.broadcasted_iota(jnp.int32, (NH, NH * HD), 0)
    hcol = lax.broadcasted_iota(jnp.int32, (NH, NH * HD), 1)
    hmask = (hcol // HD == hrow).astype(jnp.float32)

    scale = 1.0 / jnp.sqrt(jnp.float32(HD))
    ctx_rows = []
    for j in range(RB):
        kp_blk = kp_refs[j][0]
        vp_blk = vp_refs[j][0]
        qmat_h = jnp.broadcast_to(qp[j:j + 1], hmask.shape) * hmask
        s = _bdot_t(qmat_h, kp_blk) * scale
        sel = jnp.broadcast_to(selected[j:j + 1], s.shape)
        s = jnp.where(sel, s, -1e30)
        mx = jnp.max(s, axis=1, keepdims=True)
        e = jnp.where(sel, jnp.exp(s - mx), 0.0)
        attn = e / jnp.sum(e, axis=1, keepdims=True)
        c = _bdot(attn, vp_blk)
        ctx_rows.append(jnp.sum(c * hmask, axis=0, keepdims=True))
    ctx = jnp.concatenate(ctx_rows, axis=0)

    out_ref[...] = _bdot_t(ctx, ow_ref[...]) + ob_ref[...]


def kernel(patches, patch_ids, valid_mask, patch_center_gps, offsets,
           W1, b1, W2, b2, lnq_g, lnq_b, lnk_g, lnk_b, in_w, in_b, out_w, out_b):
    N, u, Din = patches.shape
    M = patch_ids.shape[0]
    D = W2.shape[0]
    hid = W1.shape[0]

    hg = int(u ** 0.5)
    dx = offsets[:, 0]
    dy = offsets[:, 1]
    i_t = jnp.clip(hg // 2 + dy, 0, hg - 1)
    j_t = jnp.clip(hg // 2 + dx, 0, hg - 1)
    idx_flat = (i_t * hg + j_t).astype(jnp.int32)
    ids = patch_ids.astype(jnp.int32)

    wq, wk, wv = in_w[:D], in_w[D:2 * D], in_w[2 * D:]
    bq, bk, bv = in_b[:D], in_b[D:2 * D], in_b[2 * D:]
    row2 = lambda v: v.reshape(1, -1)

    full2 = lambda r, c: pl.BlockSpec((r, c), lambda t: (0, 0))
    blk3 = lambda: pl.BlockSpec((PB, u, D), lambda t: (t, 0, 0))
    a3, kp3, vp3 = pl.pallas_call(
        _k1_body,
        grid=(N // PB,),
        in_specs=[
            pl.BlockSpec((PB, u, Din), lambda t: (t, 0, 0)),
            full2(hid, Din), full2(1, hid),
            full2(D, hid), full2(1, D),
            full2(1, D), full2(1, D),
            full2(D, D), full2(1, D),
            full2(D, D), full2(1, D),
        ],
        out_specs=[blk3(), blk3(), blk3()],
        out_shape=[jax.ShapeDtypeStruct((N, u, D), jnp.float32),
                   jax.ShapeDtypeStruct((N, u, D), jnp.bfloat16),
                   jax.ShapeDtypeStruct((N, u, D), jnp.bfloat16)],
    )(patches, W1, row2(b1), W2, row2(b2), row2(lnk_g), row2(lnk_b),
      wk, row2(bk), wv, row2(bv))

    def gat(j):
        return pl.BlockSpec(
            (1, u, D), lambda m, ids_r, idx_r, j=j: (ids_r[m * RB + j], 0, 0))
    cst = lambda r, c: pl.BlockSpec((r, c), lambda m, ids_r, idx_r: (0, 0))
    grid_spec = pltpu.PrefetchScalarGridSpec(
        num_scalar_prefetch=2,
        grid=(M // RB,),
        in_specs=(
            [gat(j) for j in range(RB)] * 3 +
            [cst(1, D), cst(1, D), cst(D, D), cst(1, D), cst(D, D), cst(1, D)]
        ),
        out_specs=pl.BlockSpec((RB, D), lambda m, ids_r, idx_r: (m, 0)),
    )
    attn_out = pl.pallas_call(
        _k2_body,
        grid_spec=grid_spec,
        out_shape=jax.ShapeDtypeStruct((M, D), jnp.float32),
    )(ids, idx_flat,
      *([a3] * RB), *([kp3] * RB), *([vp3] * RB),
      row2(lnq_g), row2(lnq_b), wq, row2(bq), out_w, row2(out_b))

    B, T = valid_mask.shape
    flat_mask = valid_mask.reshape(-1)
    rank = jnp.cumsum(flat_mask.astype(jnp.int32)) - 1
    placed = attn_out[jnp.clip(rank, 0, M - 1)]
    return jnp.where(flat_mask[:, None], placed,
                     jnp.zeros((), dtype=attn_out.dtype)).reshape(B, T, D)
